# baseline (speedup 1.0000x reference)
import functools

import jax
import jax.numpy as jnp
from jax import lax
from jax.experimental import pallas as pl
from jax.experimental.pallas import tpu as pltpu
from jax.experimental.pallas import tpu_sc as plsc

_NZ = 2048
_RGRID0 = -4.0
_ZGRID0 = -4.0
_H = 0.00390625
_INV_H = 256.0
_SCALE = 65536.0
_IMAX = 2046.0

_NC = 2
_NS = 16
_NW = _NC * _NS
_LANES = 16

_CHUNK = 2048
_SLICE = 128
_NSLICE = _CHUNK // _SLICE
_VPS = _SLICE // _LANES


def _corner_f(v, grid0):
    scaled = (v - grid0) * _INV_H
    return jnp.minimum(jnp.maximum(scaled, 0.0), _IMAX).astype(jnp.int32)


@jax.jit
def _run(r, z, timetable):
    n = r.shape[0]
    nchunk = n // _NW // _CHUNK
    mesh = plsc.VectorSubcoreMesh(core_axis_name="c", subcore_axis_name="s")

    @functools.partial(
        pl.kernel,
        out_type=jax.ShapeDtypeStruct((n,), jnp.float32),
        mesh=mesh,
        scratch_types=[
            pltpu.VMEM((_CHUNK,), jnp.float32),
            pltpu.VMEM((_CHUNK,), jnp.float32),
            pltpu.VMEM((4, _NSLICE, _SLICE), jnp.int32),
            pltpu.VMEM((4, _NSLICE, _SLICE), jnp.float32),
            pltpu.VMEM((_CHUNK,), jnp.float32),
            pltpu.SemaphoreType.DMA,
        ],
    )
    def body(r_hbm, z_hbm, tt_hbm, out_hbm, r_v, z_v, idx_v, q_v, o_v, sem):
        qpw = n // _NW
        wid = lax.axis_index("s") * _NC + lax.axis_index("c")
        base = wid * qpw

        @pl.loop(0, nchunk)
        def _chunk(c):
            off = base + c * _CHUNK
            pltpu.sync_copy(r_hbm.at[pl.ds(off, _CHUNK)], r_v)
            pltpu.sync_copy(z_hbm.at[pl.ds(off, _CHUNK)], z_v)

            @pl.loop(0, _NSLICE)
            def _indices(j):
                @pl.loop(0, _VPS)
                def _vreg(t):
                    i = j * _SLICE + t * _LANES
                    ir = _corner_f(r_v[pl.ds(i, _LANES)], _RGRID0)
                    iz = _corner_f(z_v[pl.ds(i, _LANES)], _ZGRID0)
                    i00 = ir * _NZ + iz
                    s = pl.ds(t * _LANES, _LANES)
                    idx_v[0, j, s] = i00
                    idx_v[1, j, s] = i00 + 1
                    idx_v[2, j, s] = i00 + _NZ
                    idx_v[3, j, s] = i00 + (_NZ + 1)

            @pl.loop(0, _NSLICE)
            def _fire(j):
                for k in range(4):
                    pltpu.async_copy(tt_hbm.at[idx_v.at[k, j]], q_v.at[k, j], sem)

            @pl.loop(0, _NSLICE)
            def _drain(j):
                for k in range(4):
                    pltpu.make_async_copy(
                        tt_hbm.at[pl.ds(0, _SLICE)], q_v.at[k, j], sem
                    ).wait()

            @pl.loop(0, _NSLICE)
            def _combine(j):
                @pl.loop(0, _VPS)
                def _vreg2(t):
                    i = j * _SLICE + t * _LANES
                    s = pl.ds(t * _LANES, _LANES)
                    rv = r_v[pl.ds(i, _LANES)]
                    zv = z_v[pl.ds(i, _LANES)]
                    irf = _corner_f(rv, _RGRID0).astype(jnp.float32)
                    izf = _corner_f(zv, _ZGRID0).astype(jnp.float32)
                    x1 = irf * _H + _RGRID0
                    x2 = (irf + 1.0) * _H + _RGRID0
                    y1 = izf * _H + _ZGRID0
                    y2 = (izf + 1.0) * _H + _ZGRID0
                    wx2 = x2 - rv
                    wx1 = rv - x1
                    wy2 = y2 - zv
                    wy1 = zv - y1
                    q11 = q_v[0, j, s]
                    q12 = q_v[1, j, s]
                    q21 = q_v[2, j, s]
                    q22 = q_v[3, j, s]
                    acc = ((q11 * wx2) * wy2 + (q21 * wx1) * wy2
                           + (q12 * wx2) * wy1 + (q22 * wx1) * wy1)
                    o_v[pl.ds(i, _LANES)] = _SCALE * acc

            pltpu.sync_copy(o_v, out_hbm.at[pl.ds(off, _CHUNK)])

    return body(r, z, timetable)


def kernel(r, z, timetable):
    return _run(r, z, timetable)
